# baseline (speedup 1.0000x reference)
import functools

import jax
import jax.numpy as jnp
from jax import lax
from jax.experimental import pallas as pl
from jax.experimental.pallas import tpu as pltpu
from jax.experimental.pallas import tpu_sc as plsc

B, S, D = 4, 4096, 1024
NC, NS, L = 2, 16, 16
NW = NC * NS
P_PER_W = S // NW
C = 4
N_CHUNKS = P_PER_W // C
NB = 4
N_GROUPS = N_CHUNKS // NB
D_VREGS = D // L


_mesh = plsc.VectorSubcoreMesh(core_axis_name="c", subcore_axis_name="s")


@functools.partial(
    pl.kernel,
    mesh=_mesh,
    out_type=jax.ShapeDtypeStruct((B, S, D), jnp.float32),
    scratch_types=[
        pltpu.VMEM((NB, C, D), jnp.float32),
        pltpu.VMEM((NB, B, C, D), jnp.float32),
        pltpu.SemaphoreType.DMA,
        pltpu.SemaphoreType.DMA,
        pltpu.SemaphoreType.DMA,
        pltpu.SemaphoreType.DMA,
        pltpu.SemaphoreType.DMA,
        pltpu.SemaphoreType.DMA,
        pltpu.SemaphoreType.DMA,
        pltpu.SemaphoreType.DMA,
    ],
)
def _pos_encode_sc(emb_hbm, pos_hbm, out_hbm, pos_v, emb_v,
                   si0, si1, si2, si3, so0, so1, so2, so3):
    wid = lax.axis_index("s") * NC + lax.axis_index("c")
    base = wid * P_PER_W
    sems_in = (si0, si1, si2, si3)
    sems_out = (so0, so1, so2, so3)

    def issue_in(ci, k):
        p0 = base + ci * C
        pltpu.async_copy(pos_hbm.at[pl.ds(p0, C)], pos_v.at[k], sems_in[k])
        pltpu.async_copy(emb_hbm.at[:, pl.ds(p0, C)], emb_v.at[k], sems_in[k])

    def wait_in(k):
        pltpu.make_async_copy(pos_hbm.at[pl.ds(0, C)], pos_v.at[k],
                              sems_in[k]).wait()
        pltpu.make_async_copy(emb_hbm.at[:, pl.ds(0, C)], emb_v.at[k],
                              sems_in[k]).wait()

    def issue_out(ci, k):
        p0 = base + ci * C
        pltpu.async_copy(emb_v.at[k], out_hbm.at[:, pl.ds(p0, C)],
                         sems_out[k])

    def wait_out(k):
        pltpu.make_async_copy(emb_v.at[k], out_hbm.at[:, pl.ds(0, C)],
                              sems_out[k]).wait()

    def compute(k):
        def row_body(r, carry):
            for j in range(D_VREGS):
                c0 = j * L
                pv = pos_v[k, r, pl.ds(c0, L)]
                for b in range(B):
                    plsc.addupdate(emb_v.at[k, b, r, pl.ds(c0, L)], pv)
            return carry

        lax.fori_loop(0, C, row_body, 0)

    issue_in(0, 0)
    issue_in(1, 1)

    def group_body(g, carry):
        for k in range(NB):
            ci = g * NB + k
            kr = (k + 2) % NB

            def refill(ci=ci, kr=kr):
                lax.cond(ci >= 2, lambda: wait_out(kr), lambda: None)
                issue_in(ci + 2, kr)

            lax.cond(ci + 2 < N_CHUNKS, refill, lambda: None)
            wait_in(k)
            compute(k)
            issue_out(ci, k)
        return carry

    lax.fori_loop(0, N_GROUPS, group_body, 0)

    for k in range(NB):
        wait_out(k)


def kernel(embeddings, pos_table):
    return _pos_encode_sc(embeddings, pos_table)


# speedup vs baseline: 1.0017x; 1.0017x over previous
import functools

import jax
import jax.numpy as jnp
from jax import lax
from jax.experimental import pallas as pl
from jax.experimental.pallas import tpu as pltpu
from jax.experimental.pallas import tpu_sc as plsc

B, S, D = 4, 4096, 1024
NC, NS, L = 2, 16, 16
NW = NC * NS
P_PER_W = S // NW
C = 8
N_CHUNKS = P_PER_W // C
NB = 3
N_GROUPS = (N_CHUNKS - 1) // NB
D_VREGS = D // L


_mesh = plsc.VectorSubcoreMesh(core_axis_name="c", subcore_axis_name="s")


@functools.partial(
    pl.kernel,
    mesh=_mesh,
    out_type=jax.ShapeDtypeStruct((B, S, D), jnp.float32),
    scratch_types=[
        pltpu.VMEM((NB, C, D), jnp.float32),
        pltpu.VMEM((NB, B, C, D), jnp.float32),
        pltpu.SemaphoreType.DMA,
        pltpu.SemaphoreType.DMA,
        pltpu.SemaphoreType.DMA,
        pltpu.SemaphoreType.DMA,
        pltpu.SemaphoreType.DMA,
        pltpu.SemaphoreType.DMA,
    ],
)
def _pos_encode_sc(emb_hbm, pos_hbm, out_hbm, pos_v, emb_v,
                   si0, si1, si2, so0, so1, so2):
    wid = lax.axis_index("s") * NC + lax.axis_index("c")
    base = wid * P_PER_W
    sems_in = (si0, si1, si2)
    sems_out = (so0, so1, so2)

    def issue_in(ci, k):
        p0 = base + ci * C
        pltpu.async_copy(pos_hbm.at[pl.ds(p0, C)], pos_v.at[k], sems_in[k])
        pltpu.async_copy(emb_hbm.at[:, pl.ds(p0, C)], emb_v.at[k], sems_in[k])

    def wait_in(k):
        pltpu.make_async_copy(pos_hbm.at[pl.ds(0, C)], pos_v.at[k],
                              sems_in[k]).wait()
        pltpu.make_async_copy(emb_hbm.at[:, pl.ds(0, C)], emb_v.at[k],
                              sems_in[k]).wait()

    def issue_out(ci, k):
        p0 = base + ci * C
        pltpu.async_copy(emb_v.at[k], out_hbm.at[:, pl.ds(p0, C)],
                         sems_out[k])

    def wait_out(k):
        pltpu.make_async_copy(emb_v.at[k], out_hbm.at[:, pl.ds(0, C)],
                              sems_out[k]).wait()

    def compute(k):
        def row_body(r, carry):
            for j in range(D_VREGS):
                c0 = j * L
                pv = pos_v[k, r, pl.ds(c0, L)]
                for b in range(B):
                    plsc.addupdate(emb_v.at[k, b, r, pl.ds(c0, L)], pv)
            return carry

        lax.fori_loop(0, C, row_body, 0)

    def step(ci, k):
        kr = (k + 2) % NB

        def refill():
            lax.cond(ci >= 1, lambda: wait_out(kr), lambda: None)
            issue_in(ci + 2, kr)

        lax.cond(ci + 2 < N_CHUNKS, refill, lambda: None)
        wait_in(k)
        compute(k)
        issue_out(ci, k)

    issue_in(0, 0)
    issue_in(1, 1)

    def group_body(g, carry):
        for k in range(NB):
            step(g * NB + k, k)
        return carry

    lax.fori_loop(0, N_GROUPS, group_body, 0)
    step(N_CHUNKS - 1, 0)

    for k in range(NB):
        wait_out(k)


def kernel(embeddings, pos_table):
    return _pos_encode_sc(embeddings, pos_table)
